# baseline (speedup 1.0000x reference)
import jax
import jax.numpy as jnp
from jax import lax
from jax.experimental import pallas as pl
from jax.experimental.pallas import tpu as pltpu
from jax.experimental.pallas import tpu_sc as plsc

B = 16384
C = 4
E = 64
V = 1000000
NC, NS, L = 2, 16, 16
NW = NC * NS
BPW = B // NW
CB = 64
NCHUNK = BPW // CB
MAXG = 128
W = 2 * E
TCOLS = V // 128
TAIL = V - TCOLS * 128
CPW = 244
XTRA = TCOLS - CPW * NW
NBLK = CPW // 2


def _transpose_body(t_nat, c_nat, t_out, c_out,
                    inb, outb, colb, tailb, si0, si1, so0, so1):
    wid = lax.axis_index("s") * NC + lax.axis_index("c")
    lanes = lax.broadcasted_iota(jnp.int32, (L,), 0)
    erows = [16 * (j % 4) + lanes for j in range(4)]
    sin = (si0, si1)
    sout = (so0, so1)
    start = wid * CPW

    def transpose_rows(ib, ob, nrows):
        def row_body(r, _):
            rr0 = jnp.full((L,), 2 * r, jnp.int32)
            rr1 = rr0 + 1
            for j in range(8):
                ob[r, pl.ds(16 * j, L)] = plsc.load_gather(
                    ib, [erows[j % 4], rr1 if j >= 4 else rr0])
            return 0
        lax.fori_loop(0, nrows, row_body, 0)

    def do_cols(nat, out):
        def issue_in(b, buf):
            c0 = (start + 2 * b) * 128
            return pltpu.async_copy(
                nat.at[:, pl.ds(c0, 256)], inb.at[buf], sin[buf])

        def wait_in(b, buf):
            c0 = (start + 2 * b) * 128
            pltpu.make_async_copy(
                nat.at[:, pl.ds(c0, 256)], inb.at[buf], sin[buf]).wait()

        def issue_out(b, buf):
            r0 = (start + 2 * b) * 64
            return pltpu.async_copy(
                outb.at[buf], out.at[pl.ds(r0, 128)], sout[buf])

        def wait_out(b, buf):
            r0 = (start + 2 * b) * 64
            pltpu.make_async_copy(
                outb.at[buf], out.at[pl.ds(r0, 128)], sout[buf]).wait()

        issue_in(jnp.int32(0), 0)
        issue_in(jnp.int32(1), 1)

        def pair_body(m, _):
            for sub in range(2):
                b = 2 * m + sub
                wait_in(b, sub)

                @pl.when(b >= 2)
                def _():
                    wait_out(b - 2, sub)
                transpose_rows(inb.at[sub], outb.at[sub], 128)
                issue_out(b, sub)

                @pl.when(b + 2 < NBLK)
                def _():
                    issue_in(b + 2, sub)
            return 0

        lax.fori_loop(0, NBLK // 2, pair_body, 0)
        wait_out(jnp.int32(NBLK - 2), 0)
        wait_out(jnp.int32(NBLK - 1), 1)

    do_cols(t_nat, t_out)
    do_cols(c_nat, c_out)

    def do_leftover(nat, out, tj):
        pltpu.async_copy(
            nat.at[:, pl.ds(tj * 128, 128)], colb, sin[0]).wait()
        transpose_rows(colb, outb.at[0], 64)
        pltpu.async_copy(
            outb.at[0, pl.ds(0, 64)], out.at[pl.ds(tj * 64, 64)],
            sout[0]).wait()

    def do_tail(nat, out):
        pltpu.async_copy(
            nat.at[:, pl.ds(TCOLS * 128, TAIL)], tailb, sin[0]).wait()
        transpose_rows(tailb, outb.at[0], TAIL // 2)
        pltpu.async_copy(
            outb.at[0, pl.ds(0, TAIL // 2)],
            out.at[pl.ds(TCOLS * 64, TAIL // 2)], sout[0]).wait()

    for i in range(XTRA):
        for which in range(2):
            @pl.when(wid == 24 + 2 * i + which)
            def _(i=i, which=which):
                do_leftover(t_nat if which == 0 else c_nat,
                            t_out if which == 0 else c_out,
                            CPW * NW + i)

    @pl.when(wid == 22)
    def _():
        do_tail(t_nat, t_out)

    @pl.when(wid == 23)
    def _():
        do_tail(c_nat, c_out)


def _gather_body(t_idx_hbm, c_idx_hbm, t_tab, c_tab, out_hbm,
                 t_idx_v, c_idx_v, t_rows, c_rows, out_v, sem0, sem1):
    wid = lax.axis_index("s") * NC + lax.axis_index("c")
    base = wid * BPW

    pltpu.sync_copy(t_idx_hbm.at[pl.ds(base, BPW)], t_idx_v.at[pl.ds(0, BPW)])
    pltpu.sync_copy(c_idx_hbm.at[pl.ds(base * C, BPW * C)],
                    c_idx_v.at[pl.ds(0, BPW * C)])
    lanes = lax.broadcasted_iota(jnp.int32, (L,), 0)
    for i in range(BPW // L):
        v = t_idx_v[pl.ds(i * L, L)]
        t_idx_v[pl.ds(i * L, L)] = lax.shift_right_logical(v, 1)
        t_idx_v[pl.ds(BPW + i * L, L)] = (v & 1) * E
    for i in range(BPW * C // L):
        p = i * L + lanes
        v = c_idx_v[pl.ds(i * L, L)]
        c_idx_v[pl.ds(i * L, L)] = lax.shift_right_logical(v, 1)
        plsc.store_scatter(
            c_idx_v, [BPW * C + (p % C) * BPW + p // C], (v & 1) * E)

    sems = (sem0, sem1)

    def issue(g, buf):
        off = g * CB
        cps = [pltpu.async_copy(
            t_tab.at[t_idx_v.at[pl.ds(off, CB)]], t_rows.at[buf], sems[buf])]
        for j in range(CB * C // MAXG):
            cps.append(pltpu.async_copy(
                c_tab.at[c_idx_v.at[pl.ds(off * C + j * MAXG, MAXG)]],
                c_rows.at[buf, pl.ds(j * MAXG, MAXG)], sems[buf]))
        return cps

    def compute(g, buf):
        tr = t_rows.at[buf]
        cr = c_rows.at[buf]
        for grp in range(CB // L):
            brow = grp * L + lanes
            tpar = t_idx_v[pl.ds(BPW + g * CB + grp * L, L)]
            cpars = [c_idx_v[pl.ds(BPW * C + c * BPW + g * CB + grp * L, L)]
                     for c in range(C)]
            def e_body(e, accs):
                tv = plsc.load_gather(tr, [brow, tpar + e])
                return tuple(
                    acc + tv * plsc.load_gather(cr, [brow * C + c, cpars[c] + e])
                    for c, acc in enumerate(accs))
            accs = lax.fori_loop(
                0, E, e_body, tuple(jnp.zeros((L,), jnp.float32)
                                    for _ in range(C)))
            for c in range(C):
                plsc.store_scatter(out_v, [brow * C + c], accs[c])
        pltpu.sync_copy(out_v, out_hbm.at[pl.ds((base + g * CB) * C, CB * C)])

    pend = issue(0, 0)
    for g in range(NCHUNK):
        nxt = issue(g + 1, (g + 1) % 2) if g + 1 < NCHUNK else None
        for cp in pend:
            cp.wait()
        compute(g, g % 2)
        pend = nxt


@jax.jit
def kernel(target, context, target_table, context_table):
    if target.ndim == 2:
        target = jnp.squeeze(target, axis=1)
    mesh = plsc.VectorSubcoreMesh(core_axis_name="c", subcore_axis_name="s")
    params = pltpu.CompilerParams(needs_layout_passes=False)

    transpose = pl.kernel(
        _transpose_body,
        out_type=(jax.ShapeDtypeStruct((V // 2, W), jnp.float32),
                  jax.ShapeDtypeStruct((V // 2, W), jnp.float32)),
        mesh=mesh,
        scratch_types=[
            pltpu.VMEM((2, E, 256), jnp.float32),
            pltpu.VMEM((2, 128, W), jnp.float32),
            pltpu.VMEM((E, 128), jnp.float32),
            pltpu.VMEM((E, TAIL), jnp.float32),
            pltpu.SemaphoreType.DMA,
            pltpu.SemaphoreType.DMA,
            pltpu.SemaphoreType.DMA,
            pltpu.SemaphoreType.DMA,
        ],
        compiler_params=params,
    )
    gather = pl.kernel(
        _gather_body,
        out_type=jax.ShapeDtypeStruct((B * C,), jnp.float32),
        mesh=mesh,
        scratch_types=[
            pltpu.VMEM((2 * BPW,), jnp.int32),
            pltpu.VMEM((2 * BPW * C,), jnp.int32),
            pltpu.VMEM((2, CB, W), jnp.float32),
            pltpu.VMEM((2, CB * C, W), jnp.float32),
            pltpu.VMEM((CB * C,), jnp.float32),
            pltpu.SemaphoreType.DMA,
            pltpu.SemaphoreType.DMA,
        ],
        compiler_params=params,
    )
    t2, c2 = transpose(jnp.swapaxes(target_table, 0, 1),
                       jnp.swapaxes(context_table, 0, 1))
    out = gather(target.astype(jnp.int32),
                 context.astype(jnp.int32).reshape(-1), t2, c2)
    return out.reshape(B, C)


# speedup vs baseline: 1.2241x; 1.2241x over previous
import jax
import jax.numpy as jnp
from jax import lax
from jax.experimental import pallas as pl
from jax.experimental.pallas import tpu as pltpu
from jax.experimental.pallas import tpu_sc as plsc

B = 16384
C = 4
E = 64
V = 1000000
NC, NS, L = 2, 16, 16
NW = NC * NS
BPW = B // NW
CB = 64
NCHUNK = BPW // CB
MAXG = 128
W = 2 * E
TCOLS = V // 128
TAIL = V - TCOLS * 128
CPW = 244
XTRA = TCOLS - CPW * NW
NBLK = CPW // 2


def _transpose_body(t_nat, c_nat, t_out, c_out,
                    inb, outb, colb, tailb, si0, si1, so0, so1):
    wid = lax.axis_index("s") * NC + lax.axis_index("c")
    lanes = lax.broadcasted_iota(jnp.int32, (L,), 0)
    sin = (si0, si1)
    sout = (so0, so1)
    start = wid * CPW
    pre_row = [8 * v + lax.shift_right_logical(lanes, 1) for v in range(16)]
    pre_col = (lanes & 1) * E

    def transpose_block(ib, ob, ncols):
        def e_body(e, _):
            col = pre_col + e
            for v in range(ncols // L):
                plsc.store_scatter(ob, [pre_row[v], col],
                                   ib[e, pl.ds(L * v, L)])
            return 0
        lax.fori_loop(0, E, e_body, 0)

    def do_cols(nat, out):
        def issue_in(b, buf):
            c0 = (start + 2 * b) * 128
            return pltpu.async_copy(
                nat.at[:, pl.ds(c0, 256)], inb.at[buf], sin[buf])

        def wait_in(b, buf):
            c0 = (start + 2 * b) * 128
            pltpu.make_async_copy(
                nat.at[:, pl.ds(c0, 256)], inb.at[buf], sin[buf]).wait()

        def issue_out(b, buf):
            r0 = (start + 2 * b) * 64
            return pltpu.async_copy(
                outb.at[buf], out.at[pl.ds(r0, 128)], sout[buf])

        def wait_out(b, buf):
            r0 = (start + 2 * b) * 64
            pltpu.make_async_copy(
                outb.at[buf], out.at[pl.ds(r0, 128)], sout[buf]).wait()

        issue_in(jnp.int32(0), 0)
        issue_in(jnp.int32(1), 1)

        def pair_body(m, _):
            for sub in range(2):
                b = 2 * m + sub
                wait_in(b, sub)

                @pl.when(b >= 2)
                def _():
                    wait_out(b - 2, sub)
                transpose_block(inb.at[sub], outb.at[sub], 256)
                issue_out(b, sub)

                @pl.when(b + 2 < NBLK)
                def _():
                    issue_in(b + 2, sub)
            return 0

        lax.fori_loop(0, NBLK // 2, pair_body, 0)
        wait_out(jnp.int32(NBLK - 2), 0)
        wait_out(jnp.int32(NBLK - 1), 1)

    do_cols(t_nat, t_out)
    do_cols(c_nat, c_out)

    def do_leftover(nat, out, tj):
        pltpu.async_copy(
            nat.at[:, pl.ds(tj * 128, 128)], colb, sin[0]).wait()
        transpose_block(colb, outb.at[0], 128)
        pltpu.async_copy(
            outb.at[0, pl.ds(0, 64)], out.at[pl.ds(tj * 64, 64)],
            sout[0]).wait()

    def do_tail(nat, out):
        pltpu.async_copy(
            nat.at[:, pl.ds(TCOLS * 128, TAIL)], tailb, sin[0]).wait()
        transpose_block(tailb, outb.at[0], TAIL)
        pltpu.async_copy(
            outb.at[0, pl.ds(0, TAIL // 2)],
            out.at[pl.ds(TCOLS * 64, TAIL // 2)], sout[0]).wait()

    for i in range(XTRA):
        for which in range(2):
            @pl.when(wid == 24 + 2 * i + which)
            def _(i=i, which=which):
                do_leftover(t_nat if which == 0 else c_nat,
                            t_out if which == 0 else c_out,
                            CPW * NW + i)

    @pl.when(wid == 22)
    def _():
        do_tail(t_nat, t_out)

    @pl.when(wid == 23)
    def _():
        do_tail(c_nat, c_out)


def _gather_body(t_idx_hbm, c_idx_hbm, t_tab, c_tab, out_hbm,
                 t_idx_v, c_idx_v, t_rows, c_rows, out_v, sem0, sem1):
    wid = lax.axis_index("s") * NC + lax.axis_index("c")
    base = wid * BPW

    pltpu.sync_copy(t_idx_hbm.at[pl.ds(base, BPW)], t_idx_v.at[pl.ds(0, BPW)])
    pltpu.sync_copy(c_idx_hbm.at[pl.ds(base * C, BPW * C)],
                    c_idx_v.at[pl.ds(0, BPW * C)])
    lanes = lax.broadcasted_iota(jnp.int32, (L,), 0)
    for i in range(BPW // L):
        v = t_idx_v[pl.ds(i * L, L)]
        t_idx_v[pl.ds(i * L, L)] = lax.shift_right_logical(v, 1)
        t_idx_v[pl.ds(BPW + i * L, L)] = (v & 1) * E
    for i in range(BPW * C // L):
        p = i * L + lanes
        v = c_idx_v[pl.ds(i * L, L)]
        c_idx_v[pl.ds(i * L, L)] = lax.shift_right_logical(v, 1)
        plsc.store_scatter(
            c_idx_v, [BPW * C + (p % C) * BPW + p // C], (v & 1) * E)

    sems = (sem0, sem1)

    def issue(g, buf):
        off = g * CB
        cps = [pltpu.async_copy(
            t_tab.at[t_idx_v.at[pl.ds(off, CB)]], t_rows.at[buf], sems[buf])]
        for j in range(CB * C // MAXG):
            cps.append(pltpu.async_copy(
                c_tab.at[c_idx_v.at[pl.ds(off * C + j * MAXG, MAXG)]],
                c_rows.at[buf, pl.ds(j * MAXG, MAXG)], sems[buf]))
        return cps

    def compute(g, buf):
        tr = t_rows.at[buf]
        cr = c_rows.at[buf]
        for grp in range(CB // L):
            brow = grp * L + lanes
            tpar = t_idx_v[pl.ds(BPW + g * CB + grp * L, L)]
            cpars = [c_idx_v[pl.ds(BPW * C + c * BPW + g * CB + grp * L, L)]
                     for c in range(C)]
            def e_body(e, accs):
                tv = plsc.load_gather(tr, [brow, tpar + e])
                return tuple(
                    acc + tv * plsc.load_gather(cr, [brow * C + c, cpars[c] + e])
                    for c, acc in enumerate(accs))
            accs = lax.fori_loop(
                0, E, e_body, tuple(jnp.zeros((L,), jnp.float32)
                                    for _ in range(C)))
            for c in range(C):
                plsc.store_scatter(out_v, [brow * C + c], accs[c])
        pltpu.sync_copy(out_v, out_hbm.at[pl.ds((base + g * CB) * C, CB * C)])

    pend = issue(0, 0)
    for g in range(NCHUNK):
        nxt = issue(g + 1, (g + 1) % 2) if g + 1 < NCHUNK else None
        for cp in pend:
            cp.wait()
        compute(g, g % 2)
        pend = nxt


@jax.jit
def kernel(target, context, target_table, context_table):
    if target.ndim == 2:
        target = jnp.squeeze(target, axis=1)
    mesh = plsc.VectorSubcoreMesh(core_axis_name="c", subcore_axis_name="s")
    params = pltpu.CompilerParams(needs_layout_passes=False)

    transpose = pl.kernel(
        _transpose_body,
        out_type=(jax.ShapeDtypeStruct((V // 2, W), jnp.float32),
                  jax.ShapeDtypeStruct((V // 2, W), jnp.float32)),
        mesh=mesh,
        scratch_types=[
            pltpu.VMEM((2, E, 256), jnp.float32),
            pltpu.VMEM((2, 128, W), jnp.float32),
            pltpu.VMEM((E, 128), jnp.float32),
            pltpu.VMEM((E, TAIL), jnp.float32),
            pltpu.SemaphoreType.DMA,
            pltpu.SemaphoreType.DMA,
            pltpu.SemaphoreType.DMA,
            pltpu.SemaphoreType.DMA,
        ],
        compiler_params=params,
    )
    gather = pl.kernel(
        _gather_body,
        out_type=jax.ShapeDtypeStruct((B * C,), jnp.float32),
        mesh=mesh,
        scratch_types=[
            pltpu.VMEM((2 * BPW,), jnp.int32),
            pltpu.VMEM((2 * BPW * C,), jnp.int32),
            pltpu.VMEM((2, CB, W), jnp.float32),
            pltpu.VMEM((2, CB * C, W), jnp.float32),
            pltpu.VMEM((CB * C,), jnp.float32),
            pltpu.SemaphoreType.DMA,
            pltpu.SemaphoreType.DMA,
        ],
        compiler_params=params,
    )
    t2, c2 = transpose(jnp.swapaxes(target_table, 0, 1),
                       jnp.swapaxes(context_table, 0, 1))
    out = gather(target.astype(jnp.int32),
                 context.astype(jnp.int32).reshape(-1), t2, c2)
    return out.reshape(B, C)
